# baseline (speedup 1.0000x reference)
import functools

import jax
import jax.numpy as jnp
from jax import lax
from jax.experimental import pallas as pl
from jax.experimental.pallas import tpu as pltpu
from jax.experimental.pallas import tpu_sc as plsc

B, S, D, R = 16384, 50, 32, 1000000
NC, NS = 2, 16
NW = NC * NS
RJF = 7812
DUMMY = RJF * 32 + 32
PJ = DUMMY + 32
APW = 245
UBPW = (S * 128) // NW

_mesh = plsc.VectorSubcoreMesh(core_axis_name="c", subcore_axis_name="s")
_params = pltpu.CompilerParams(use_tc_tiling_on_sc=True, needs_layout_passes=False)


@functools.partial(
    pl.kernel,
    mesh=_mesh,
    out_type=jax.ShapeDtypeStruct((PJ, 128), jnp.float32),
    compiler_params=_params,
    scratch_types=[
        pltpu.VMEM((32, 128), jnp.float32),
        pltpu.VMEM((32, 128), jnp.float32),
        pltpu.VMEM((32, 128), jnp.float32),
        pltpu.VMEM((32, 128), jnp.float32),
        pltpu.SemaphoreType.DMA,
        pltpu.SemaphoreType.DMA,
        pltpu.SemaphoreType.DMA,
        pltpu.SemaphoreType.DMA,
    ],
)
def _pack(wt_hbm, p_hbm, t0, t1, p0, p1, ain0, ain1, aout0, aout1):
    wid = lax.axis_index("s") * NC + lax.axis_index("c")
    lanes = lax.iota(jnp.int32, 16)
    rowbase = lax.shift_right_logical(lanes, 2)
    colbase = lax.shift_left(lax.bitwise_and(lanes, jnp.int32(3)), 5)
    tiles = (t0, t1)
    packs = (p0, p1)
    ains = (ain0, ain1)
    aouts = (aout0, aout1)

    def src_j(n):
        jj = wid + NW * n
        return jnp.where(jj < RJF, jj, 0), jj < RJF

    def fire_in(n, par):
        j, _ = src_j(n)
        for i in range(4):
            pltpu.async_copy(
                wt_hbm.at[pl.ds(i * 8, 8),
                          pl.ds(pl.multiple_of(j * 128, 128), 128)],
                tiles[par].at[pl.ds(i * 8, 8)],
                ains[par],
            )

    def drain_in(par):
        for i in range(4):
            pltpu.make_async_copy(
                wt_hbm.at[pl.ds(i * 8, 8), pl.ds(0, 128)],
                tiles[par].at[pl.ds(i * 8, 8)],
                ains[par],
            ).wait()

    rows8 = tuple(rowbase + 4 * g for g in range(8))

    def scatter(par):
        tl = tiles[par]
        pk = packs[par]
        for d in range(32):
            cols = colbase + d
            for g in range(8):
                v = tl[d, pl.ds(g * 16, 16)]
                plsc.store_scatter(pk, [rows8[g], cols], v)

    def fire_out(n, par):
        j, valid = src_j(n)
        dst = jnp.where(valid, j * 32, DUMMY)
        pltpu.async_copy(
            packs[par], p_hbm.at[pl.ds(pl.multiple_of(dst, 8), 32)],
            aouts[par]
        )

    def drain_out(par):
        pltpu.make_async_copy(
            packs[par], p_hbm.at[pl.ds(0, 32)], aouts[par]
        ).wait()

    fire_in(0, 0)

    def body(m, carry):
        n0 = 2 * m
        fire_in(n0 + 1, 1)
        drain_in(0)

        @pl.when(m > 0)
        def _():
            drain_out(0)

        scatter(0)
        fire_out(n0, 0)
        fire_in(n0 + 2, 0)
        drain_in(1)

        @pl.when(m > 0)
        def _():
            drain_out(1)

        scatter(1)
        fire_out(n0 + 1, 1)
        return carry

    lax.fori_loop(0, APW // 2, body, 0)
    drain_in(0)
    drain_out(0)
    scatter(0)
    fire_out(APW - 1, 0)
    drain_out(1)
    drain_out(0)

    @pl.when(wid == 0)
    def _():
        for i in range(4):
            pltpu.async_copy(
                wt_hbm.at[pl.ds(i * 8, 8),
                          pl.ds(pl.multiple_of(RJF * 128, 128), 128)],
                t0.at[pl.ds(i * 8, 8)],
                ain0,
            )
        for i in range(4):
            pltpu.make_async_copy(
                wt_hbm.at[pl.ds(i * 8, 8), pl.ds(0, 128)],
                t0.at[pl.ds(i * 8, 8)],
                ain0,
            ).wait()
        for d in range(32):
            cols = colbase + d
            for g in range(4):
                v = t0[d, pl.ds(g * 16, 16)]
                plsc.store_scatter(p0, [rows8[g], cols], v)
        pltpu.async_copy(
            p0.at[pl.ds(0, 16)], p_hbm.at[pl.ds(RJF * 32, 16)], aout0
        )
        pltpu.make_async_copy(
            p0.at[pl.ds(0, 16)], p_hbm.at[pl.ds(0, 16)], aout0
        ).wait()


@functools.partial(
    pl.kernel,
    mesh=_mesh,
    out_type=jax.ShapeDtypeStruct((S, D, B), jnp.float32),
    compiler_params=_params,
    scratch_types=[
        pltpu.VMEM((128,), jnp.int32),
        pltpu.VMEM((128,), jnp.int32),
        pltpu.VMEM((128,), jnp.int32),
        pltpu.VMEM((128,), jnp.int32),
        pltpu.VMEM((128,), jnp.int32),
        pltpu.VMEM((128,), jnp.int32),
        pltpu.VMEM((128, 128), jnp.float32),
        pltpu.VMEM((128, 128), jnp.float32),
        pltpu.VMEM((32, 128), jnp.float32),
        pltpu.VMEM((32, 128), jnp.float32),
        pltpu.SemaphoreType.DMA,
        pltpu.SemaphoreType.DMA,
        pltpu.SemaphoreType.DMA,
        pltpu.SemaphoreType.DMA,
        pltpu.SemaphoreType.DMA,
        pltpu.SemaphoreType.DMA,
    ],
)
def _gather(tt_hbm, p_hbm, out_hbm, i0, i1, f0, f1, c0, c1, st0, st1,
            ob0, ob1, si0, si1, sg0, sg1, so0, so1):
    wid = lax.axis_index("s") * NC + lax.axis_index("c")
    idxs = (i0, i1)
    j4s = (f0, f1)
    cbs = (c0, c1)
    stages = (st0, st1)
    outbs = (ob0, ob1)
    sis = (si0, si1)
    sgs = (sg0, sg1)
    sos = (so0, so1)
    lanes = lax.iota(jnp.int32, 16)
    lanes16 = tuple(lanes + g * 16 for g in range(8))

    def ub_sj(n):
        ub = wid * UBPW + n
        return ub // 128, ub % 128

    def fire_idx(n, par):
        s, j = ub_sj(n)
        pltpu.async_copy(
            tt_hbm.at[s, pl.ds(pl.multiple_of(j * 128, 128), 128)],
            idxs[par], sis[par]
        )

    def drain_idx(par):
        pltpu.make_async_copy(
            tt_hbm.at[0, pl.ds(0, 128)], idxs[par], sis[par]
        ).wait()

    def prep(n, par):
        for g in range(8):
            t = idxs[par][pl.ds(g * 16, 16)]
            j4s[par][pl.ds(g * 16, 16)] = lax.shift_right_logical(t, 2)
            cbs[par][pl.ds(g * 16, 16)] = lax.shift_left(
                lax.bitwise_and(t, jnp.int32(3)), 5)

    def fire_gather(par):
        pltpu.async_copy(p_hbm.at[j4s[par]], stages[par], sgs[par])

    def drain_gather(par):
        pltpu.make_async_copy(
            p_hbm.at[j4s[par]], stages[par], sgs[par]
        ).wait()

    def extract(par):
        st = stages[par]
        ob = outbs[par]
        cb = cbs[par]
        for g in range(8):
            rows = lanes16[g]
            csub = cb[pl.ds(g * 16, 16)]
            for d in range(32):
                v = plsc.load_gather(st, [rows, csub + d])
                ob[d, pl.ds(g * 16, 16)] = v

    def fire_out(n, par):
        s, j = ub_sj(n)
        pltpu.async_copy(
            outbs[par],
            out_hbm.at[s, pl.ds(0, 32),
                       pl.ds(pl.multiple_of(j * 128, 128), 128)],
            sos[par],
        )

    def drain_out(par):
        pltpu.make_async_copy(
            outbs[par], out_hbm.at[0, pl.ds(0, 32), pl.ds(0, 128)], sos[par]
        ).wait()

    fire_idx(0, 0)
    fire_idx(1, 1)
    drain_idx(0)
    prep(0, 0)
    fire_gather(0)

    def body(m, carry):
        n0 = 2 * m
        drain_idx(1)
        prep(n0 + 1, 1)

        @pl.when(n0 + 2 < UBPW)
        def _():
            fire_idx(n0 + 2, 0)

        fire_gather(1)
        drain_gather(0)

        @pl.when(m > 0)
        def _():
            drain_out(0)

        extract(0)
        fire_out(n0, 0)

        @pl.when(n0 + 2 < UBPW)
        def _():
            drain_idx(0)
            prep(n0 + 2, 0)
            fire_gather(0)

        @pl.when(n0 + 3 < UBPW)
        def _():
            fire_idx(n0 + 3, 1)

        drain_gather(1)

        @pl.when(m > 0)
        def _():
            drain_out(1)

        extract(1)
        fire_out(n0 + 1, 1)
        return carry

    lax.fori_loop(0, UBPW // 2, body, 0)
    drain_out(0)
    drain_out(1)


def kernel(token_ids, weight):
    tT = token_ids.T.astype(jnp.int32)
    wT = weight.T
    p = _pack(wT)
    out = _gather(tT, p)
    return out.transpose(2, 0, 1)


# speedup vs baseline: 1.5782x; 1.5782x over previous
import functools

import jax
import jax.numpy as jnp
from jax import lax
from jax.experimental import pallas as pl
from jax.experimental.pallas import tpu as pltpu
from jax.experimental.pallas import tpu_sc as plsc

B, S, D, R = 16384, 50, 32, 1000000
NC, NS = 2, 16
NW = NC * NS
RJF = 7812
DUMMY = RJF * 32 + 32
PJ = DUMMY + 32
APW = 245
UBPW = (S * 128) // NW

_mesh = plsc.VectorSubcoreMesh(core_axis_name="c", subcore_axis_name="s")
_params = pltpu.CompilerParams(use_tc_tiling_on_sc=True, needs_layout_passes=False)


@functools.partial(
    pl.kernel,
    mesh=_mesh,
    out_type=jax.ShapeDtypeStruct((PJ, 128), jnp.float32),
    compiler_params=_params,
    scratch_types=[
        pltpu.VMEM((32, 128), jnp.float32),
        pltpu.VMEM((32, 128), jnp.float32),
        pltpu.VMEM((32, 128), jnp.float32),
        pltpu.VMEM((32, 128), jnp.float32),
        pltpu.SemaphoreType.DMA,
        pltpu.SemaphoreType.DMA,
        pltpu.SemaphoreType.DMA,
        pltpu.SemaphoreType.DMA,
    ],
)
def _pack(wt_hbm, p_hbm, t0, t1, p0, p1, ain0, ain1, aout0, aout1):
    wid = lax.axis_index("s") * NC + lax.axis_index("c")
    lanes = lax.iota(jnp.int32, 16)
    rowbase = lax.shift_right_logical(lanes, 2)
    colbase = lax.shift_left(lax.bitwise_and(lanes, jnp.int32(3)), 5)
    tiles = (t0, t1)
    packs = (p0, p1)
    ains = (ain0, ain1)
    aouts = (aout0, aout1)

    def src_j(n):
        jj = wid + NW * n
        return jnp.where(jj < RJF, jj, 0), jj < RJF

    def fire_in(n, par):
        j, _ = src_j(n)
        for i in range(4):
            pltpu.async_copy(
                wt_hbm.at[pl.ds(i * 8, 8),
                          pl.ds(pl.multiple_of(j * 128, 128), 128)],
                tiles[par].at[pl.ds(i * 8, 8)],
                ains[par],
            )

    def drain_in(par):
        for i in range(4):
            pltpu.make_async_copy(
                wt_hbm.at[pl.ds(i * 8, 8), pl.ds(0, 128)],
                tiles[par].at[pl.ds(i * 8, 8)],
                ains[par],
            ).wait()

    rows8 = tuple(rowbase + 4 * g for g in range(8))

    def scatter(par):
        tl = tiles[par]
        pk = packs[par]

        @plsc.parallel_loop(0, 32, unroll=4)
        def _(d):
            cols = colbase + d
            for g in range(8):
                v = tl[d, pl.ds(g * 16, 16)]
                plsc.store_scatter(pk, [rows8[g], cols], v)

    def fire_out(n, par):
        j, valid = src_j(n)
        dst = jnp.where(valid, j * 32, DUMMY)
        pltpu.async_copy(
            packs[par], p_hbm.at[pl.ds(pl.multiple_of(dst, 8), 32)],
            aouts[par]
        )

    def drain_out(par):
        pltpu.make_async_copy(
            packs[par], p_hbm.at[pl.ds(0, 32)], aouts[par]
        ).wait()

    fire_in(0, 0)

    def body(m, carry):
        n0 = 2 * m
        fire_in(n0 + 1, 1)
        drain_in(0)

        @pl.when(m > 0)
        def _():
            drain_out(0)

        scatter(0)
        fire_out(n0, 0)
        fire_in(n0 + 2, 0)
        drain_in(1)

        @pl.when(m > 0)
        def _():
            drain_out(1)

        scatter(1)
        fire_out(n0 + 1, 1)
        return carry

    lax.fori_loop(0, APW // 2, body, 0)
    drain_in(0)
    drain_out(0)
    scatter(0)
    fire_out(APW - 1, 0)
    drain_out(1)
    drain_out(0)

    @pl.when(wid == 0)
    def _():
        for i in range(4):
            pltpu.async_copy(
                wt_hbm.at[pl.ds(i * 8, 8),
                          pl.ds(pl.multiple_of(RJF * 128, 128), 128)],
                t0.at[pl.ds(i * 8, 8)],
                ain0,
            )
        for i in range(4):
            pltpu.make_async_copy(
                wt_hbm.at[pl.ds(i * 8, 8), pl.ds(0, 128)],
                t0.at[pl.ds(i * 8, 8)],
                ain0,
            ).wait()
        for d in range(32):
            cols = colbase + d
            for g in range(4):
                v = t0[d, pl.ds(g * 16, 16)]
                plsc.store_scatter(p0, [rows8[g], cols], v)
        pltpu.async_copy(
            p0.at[pl.ds(0, 16)], p_hbm.at[pl.ds(RJF * 32, 16)], aout0
        )
        pltpu.make_async_copy(
            p0.at[pl.ds(0, 16)], p_hbm.at[pl.ds(0, 16)], aout0
        ).wait()


@functools.partial(
    pl.kernel,
    mesh=_mesh,
    out_type=jax.ShapeDtypeStruct((S, D, B), jnp.float32),
    compiler_params=_params,
    scratch_types=[
        pltpu.VMEM((128,), jnp.int32),
        pltpu.VMEM((128,), jnp.int32),
        pltpu.VMEM((128,), jnp.int32),
        pltpu.VMEM((128,), jnp.int32),
        pltpu.VMEM((128,), jnp.int32),
        pltpu.VMEM((128,), jnp.int32),
        pltpu.VMEM((128, 128), jnp.float32),
        pltpu.VMEM((128, 128), jnp.float32),
        pltpu.VMEM((32, 128), jnp.float32),
        pltpu.VMEM((32, 128), jnp.float32),
        pltpu.SemaphoreType.DMA,
        pltpu.SemaphoreType.DMA,
        pltpu.SemaphoreType.DMA,
        pltpu.SemaphoreType.DMA,
        pltpu.SemaphoreType.DMA,
        pltpu.SemaphoreType.DMA,
    ],
)
def _gather(tt_hbm, p_hbm, out_hbm, i0, i1, f0, f1, c0, c1, st0, st1,
            ob0, ob1, si0, si1, sg0, sg1, so0, so1):
    wid = lax.axis_index("s") * NC + lax.axis_index("c")
    idxs = (i0, i1)
    j4s = (f0, f1)
    cbs = (c0, c1)
    stages = (st0, st1)
    outbs = (ob0, ob1)
    sis = (si0, si1)
    sgs = (sg0, sg1)
    sos = (so0, so1)
    lanes = lax.iota(jnp.int32, 16)
    lanes16 = tuple(lanes + g * 16 for g in range(8))

    def ub_sj(n):
        ub = wid * UBPW + n
        return ub // 128, ub % 128

    def fire_idx(n, par):
        s, j = ub_sj(n)
        pltpu.async_copy(
            tt_hbm.at[s, pl.ds(pl.multiple_of(j * 128, 128), 128)],
            idxs[par], sis[par]
        )

    def drain_idx(par):
        pltpu.make_async_copy(
            tt_hbm.at[0, pl.ds(0, 128)], idxs[par], sis[par]
        ).wait()

    def prep(n, par):
        for g in range(8):
            t = idxs[par][pl.ds(g * 16, 16)]
            j4s[par][pl.ds(g * 16, 16)] = lax.shift_right_logical(t, 2)
            cbs[par][pl.ds(g * 16, 16)] = lax.shift_left(
                lax.bitwise_and(t, jnp.int32(3)), 5)

    def fire_gather(par):
        pltpu.async_copy(p_hbm.at[j4s[par]], stages[par], sgs[par])

    def drain_gather(par):
        pltpu.make_async_copy(
            p_hbm.at[j4s[par]], stages[par], sgs[par]
        ).wait()

    def extract(par):
        st = stages[par]
        ob = outbs[par]
        cb = cbs[par]
        for g in range(8):
            rows = lanes16[g]
            csub = cb[pl.ds(g * 16, 16)]

            @plsc.parallel_loop(0, 32, unroll=4)
            def _(d, _rows=rows, _csub=csub, _g=g):
                v = plsc.load_gather(st, [_rows, _csub + d])
                ob[d, pl.ds(_g * 16, 16)] = v

    def fire_out(n, par):
        s, j = ub_sj(n)
        pltpu.async_copy(
            outbs[par],
            out_hbm.at[s, pl.ds(0, 32),
                       pl.ds(pl.multiple_of(j * 128, 128), 128)],
            sos[par],
        )

    def drain_out(par):
        pltpu.make_async_copy(
            outbs[par], out_hbm.at[0, pl.ds(0, 32), pl.ds(0, 128)], sos[par]
        ).wait()

    fire_idx(0, 0)
    fire_idx(1, 1)
    drain_idx(0)
    prep(0, 0)
    fire_gather(0)

    def body(m, carry):
        n0 = 2 * m
        drain_idx(1)
        prep(n0 + 1, 1)

        @pl.when(n0 + 2 < UBPW)
        def _():
            fire_idx(n0 + 2, 0)

        fire_gather(1)
        drain_gather(0)

        @pl.when(m > 0)
        def _():
            drain_out(0)

        extract(0)
        fire_out(n0, 0)

        @pl.when(n0 + 2 < UBPW)
        def _():
            drain_idx(0)
            prep(n0 + 2, 0)
            fire_gather(0)

        @pl.when(n0 + 3 < UBPW)
        def _():
            fire_idx(n0 + 3, 1)

        drain_gather(1)

        @pl.when(m > 0)
        def _():
            drain_out(1)

        extract(1)
        fire_out(n0 + 1, 1)
        return carry

    lax.fori_loop(0, UBPW // 2, body, 0)
    drain_out(0)
    drain_out(1)


def kernel(token_ids, weight):
    tT = token_ids.T.astype(jnp.int32)
    wT = weight.T
    p = _pack(wT)
    out = _gather(tT, p)
    return out.transpose(2, 0, 1)
